# baseline (speedup 1.0000x reference)
import numpy as np
import jax
import jax.numpy as jnp
from jax.experimental import pallas as pl
from jax.experimental.pallas import tpu as pltpu

_EPS = 1e-08
_LOG_EPS = float(np.log(1e-08))
_N_SLOTS = 16
_S = 2


def _sb_kernel(feat_ref, inw_ref, inb_ref, outw_ref, outb_ref, lseed_ref,
               out_ref, g_ref, masks_ref):
    P = feat_ref.shape[1]
    inw_bf = inw_ref[...].astype(jnp.bfloat16)
    ids = jax.lax.broadcasted_iota(jnp.int32, (1, P), 1)

    fs, projs = [], []
    for s in range(_S):
        f = feat_ref[s]
        fs.append(f)
        proj = jax.lax.dot_general(f.astype(jnp.bfloat16), inw_bf,
                                   (((1,), (1,)), ((), ())),
                                   preferred_element_type=jnp.float32)
        proj = proj + inb_ref[...]
        norm = jnp.sqrt(jnp.sum(proj * proj, axis=-1, keepdims=True))
        proj = proj / jnp.maximum(norm, 1e-12)
        projs.append(proj)
        g_ref[s] = jax.lax.dot_general(proj, proj, (((1,), (1,)), ((), ())),
                                       preferred_element_type=jnp.float32,
                                       precision=jax.lax.Precision.HIGHEST)

    scopes = [jnp.zeros((1, P), jnp.float32)] * _S
    for k in range(_N_SLOTS):
        for s in range(_S):
            log_scope = scopes[s]
            v = log_scope + lseed_ref[s]
            mx = jnp.max(v)
            idx = jnp.min(jnp.where(v == mx, ids, P))
            grow = g_ref[s, pl.ds(idx, 1), :]
            dists = 2.0 - 2.0 * grow
            log_alpha = jnp.maximum(-dists, _LOG_EPS)
            masks_ref[s, k:k + 1, :] = jnp.exp(log_scope + log_alpha)
            scopes[s] = log_scope + jnp.log(
                jnp.maximum(1.0 - jnp.exp(log_alpha), _EPS))

    outw_bf = outw_ref[...].astype(jnp.bfloat16)
    for s in range(_S):
        masks = masks_ref[s]
        slots = jax.lax.dot_general(masks.astype(jnp.bfloat16),
                                    fs[s].astype(jnp.bfloat16),
                                    (((1,), (0,)), ((), ())),
                                    preferred_element_type=jnp.float32)
        msum = jnp.sum(masks, axis=1, keepdims=True)
        slots = slots / jnp.maximum(msum, _EPS)
        outv = jax.lax.dot_general(slots.astype(jnp.bfloat16), outw_bf,
                                   (((1,), (1,)), ((), ())),
                                   preferred_element_type=jnp.float32)
        out_ref[s] = outv + outb_ref[...]


def kernel(features, in_w, in_b, out_w, out_b):
    bs, P, D = features.shape
    O = out_w.shape[0]
    seeds = jax.random.uniform(jax.random.key(42), (bs, P), dtype=jnp.float32)
    log_seeds = jnp.log(jnp.clip(seeds, _EPS, None)).reshape(bs, 1, P)
    in_b2 = in_b.reshape(1, D)
    out_b2 = out_b.reshape(1, O)
    return pl.pallas_call(
        _sb_kernel,
        grid=(bs // _S,),
        in_specs=[
            pl.BlockSpec((_S, P, D), lambda b: (b, 0, 0)),
            pl.BlockSpec((D, D), lambda b: (0, 0)),
            pl.BlockSpec((1, D), lambda b: (0, 0)),
            pl.BlockSpec((O, D), lambda b: (0, 0)),
            pl.BlockSpec((1, O), lambda b: (0, 0)),
            pl.BlockSpec((_S, 1, P), lambda b: (b, 0, 0)),
        ],
        out_specs=pl.BlockSpec((_S, _N_SLOTS, O), lambda b: (b, 0, 0)),
        out_shape=jax.ShapeDtypeStruct((bs, _N_SLOTS, O), jnp.float32),
        scratch_shapes=[
            pltpu.VMEM((_S, P, P), jnp.float32),
            pltpu.VMEM((_S, _N_SLOTS, P), jnp.float32),
        ],
        compiler_params=pltpu.CompilerParams(
            dimension_semantics=("parallel",)),
    )(features, in_w, in_b2, out_w, out_b2, log_seeds)


# speedup vs baseline: 1.2617x; 1.2617x over previous
import numpy as np
import jax
import jax.numpy as jnp
from jax.experimental import pallas as pl
from jax.experimental.pallas import tpu as pltpu

_EPS = 1e-08
_LOG_EPS = float(np.log(1e-08))
_N_SLOTS = 16
_S = 4


def _sb_kernel(feat_ref, inw_ref, inb_ref, outw_ref, outb_ref, lseed_ref,
               out_ref, g_ref, masks_ref):
    P = feat_ref.shape[1]
    inw_bf = inw_ref[...].astype(jnp.bfloat16)
    ids = jax.lax.broadcasted_iota(jnp.int32, (1, P), 1)

    fs, projs = [], []
    for s in range(_S):
        f = feat_ref[s]
        fs.append(f)
        proj = jax.lax.dot_general(f.astype(jnp.bfloat16), inw_bf,
                                   (((1,), (1,)), ((), ())),
                                   preferred_element_type=jnp.float32)
        proj = proj + inb_ref[...]
        norm = jnp.sqrt(jnp.sum(proj * proj, axis=-1, keepdims=True))
        proj = proj / jnp.maximum(norm, 1e-12)
        projs.append(proj)
        hi = proj.astype(jnp.bfloat16)
        lo = (proj - hi.astype(jnp.float32)).astype(jnp.bfloat16)
        dims = (((1,), (1,)), ((), ()))
        g = jax.lax.dot_general(hi, hi, dims,
                                preferred_element_type=jnp.float32)
        g = g + jax.lax.dot_general(hi, lo, dims,
                                    preferred_element_type=jnp.float32)
        g = g + jax.lax.dot_general(lo, hi, dims,
                                    preferred_element_type=jnp.float32)
        g = g + jax.lax.dot_general(lo, lo, dims,
                                    preferred_element_type=jnp.float32)
        g_ref[s] = g

    scopes = [jnp.zeros((1, P), jnp.float32)] * _S
    for k in range(_N_SLOTS):
        for s in range(_S):
            log_scope = scopes[s]
            v = log_scope + lseed_ref[s]
            mx = jnp.max(v)
            idx = jnp.min(jnp.where(v == mx, ids, P))
            grow = g_ref[s, pl.ds(idx, 1), :]
            dists = 2.0 - 2.0 * grow
            log_alpha = jnp.maximum(-dists, _LOG_EPS)
            masks_ref[s, k:k + 1, :] = jnp.exp(log_scope + log_alpha)
            scopes[s] = log_scope + jnp.log(
                jnp.maximum(1.0 - jnp.exp(log_alpha), _EPS))

    outw_bf = outw_ref[...].astype(jnp.bfloat16)
    for s in range(_S):
        masks = masks_ref[s]
        slots = jax.lax.dot_general(masks.astype(jnp.bfloat16),
                                    fs[s].astype(jnp.bfloat16),
                                    (((1,), (0,)), ((), ())),
                                    preferred_element_type=jnp.float32)
        msum = jnp.sum(masks, axis=1, keepdims=True)
        slots = slots / jnp.maximum(msum, _EPS)
        outv = jax.lax.dot_general(slots.astype(jnp.bfloat16), outw_bf,
                                   (((1,), (1,)), ((), ())),
                                   preferred_element_type=jnp.float32)
        out_ref[s] = outv + outb_ref[...]


def kernel(features, in_w, in_b, out_w, out_b):
    bs, P, D = features.shape
    O = out_w.shape[0]
    seeds = jax.random.uniform(jax.random.key(42), (bs, P), dtype=jnp.float32)
    log_seeds = jnp.log(jnp.clip(seeds, _EPS, None)).reshape(bs, 1, P)
    in_b2 = in_b.reshape(1, D)
    out_b2 = out_b.reshape(1, O)
    return pl.pallas_call(
        _sb_kernel,
        grid=(bs // _S,),
        in_specs=[
            pl.BlockSpec((_S, P, D), lambda b: (b, 0, 0)),
            pl.BlockSpec((D, D), lambda b: (0, 0)),
            pl.BlockSpec((1, D), lambda b: (0, 0)),
            pl.BlockSpec((O, D), lambda b: (0, 0)),
            pl.BlockSpec((1, O), lambda b: (0, 0)),
            pl.BlockSpec((_S, 1, P), lambda b: (b, 0, 0)),
        ],
        out_specs=pl.BlockSpec((_S, _N_SLOTS, O), lambda b: (b, 0, 0)),
        out_shape=jax.ShapeDtypeStruct((bs, _N_SLOTS, O), jnp.float32),
        scratch_shapes=[
            pltpu.VMEM((_S, P, P), jnp.float32),
            pltpu.VMEM((_S, _N_SLOTS, P), jnp.float32),
        ],
        compiler_params=pltpu.CompilerParams(
            dimension_semantics=("parallel",)),
    )(features, in_w, in_b2, out_w, out_b2, log_seeds)


# speedup vs baseline: 1.4474x; 1.1473x over previous
import numpy as np
import jax
import jax.numpy as jnp
from jax.experimental import pallas as pl
from jax.experimental.pallas import tpu as pltpu

_EPS = 1e-08
_LOG_EPS = float(np.log(1e-08))
_N_SLOTS = 16
_S = 4


def _sb_kernel(feat_ref, inw_ref, inb_ref, outw_ref, outb_ref, lseed_ref,
               out_ref, g_ref, masks_ref):
    P = feat_ref.shape[1]
    inw_bf = inw_ref[...].astype(jnp.bfloat16)
    ids = jax.lax.broadcasted_iota(jnp.int32, (1, P), 1)

    fs, projs = [], []
    for s in range(_S):
        f = feat_ref[s]
        fs.append(f)
        proj = jax.lax.dot_general(f.astype(jnp.bfloat16), inw_bf,
                                   (((1,), (1,)), ((), ())),
                                   preferred_element_type=jnp.float32)
        proj = proj + inb_ref[...]
        norm = jnp.sqrt(jnp.sum(proj * proj, axis=-1, keepdims=True))
        proj = proj / jnp.maximum(norm, 1e-12)
        projs.append(proj)
        hi = proj.astype(jnp.bfloat16)
        lo = (proj - hi.astype(jnp.float32)).astype(jnp.bfloat16)
        u = jnp.concatenate([hi, lo, hi, lo], axis=1)
        w = jnp.concatenate([hi, lo, lo, hi], axis=1)
        g_ref[s] = jax.lax.dot_general(u, w, (((1,), (1,)), ((), ())),
                                       preferred_element_type=jnp.float32)

    scopes = [jnp.zeros((1, P), jnp.float32)] * _S
    for k in range(_N_SLOTS):
        for s in range(_S):
            log_scope = scopes[s]
            v = log_scope + lseed_ref[s]
            mx = jnp.max(v)
            idx = jnp.min(jnp.where(v == mx, ids, P))
            grow = g_ref[s, pl.ds(idx, 1), :]
            dists = 2.0 - 2.0 * grow
            log_alpha = jnp.maximum(-dists, _LOG_EPS)
            masks_ref[s, k:k + 1, :] = jnp.exp(log_scope + log_alpha)
            scopes[s] = log_scope + jnp.log(
                jnp.maximum(1.0 - jnp.exp(log_alpha), _EPS))

    outw_bf = outw_ref[...].astype(jnp.bfloat16)
    for s in range(_S):
        masks = masks_ref[s]
        slots = jax.lax.dot_general(masks.astype(jnp.bfloat16),
                                    fs[s].astype(jnp.bfloat16),
                                    (((1,), (0,)), ((), ())),
                                    preferred_element_type=jnp.float32)
        msum = jnp.sum(masks, axis=1, keepdims=True)
        slots = slots / jnp.maximum(msum, _EPS)
        outv = jax.lax.dot_general(slots.astype(jnp.bfloat16), outw_bf,
                                   (((1,), (1,)), ((), ())),
                                   preferred_element_type=jnp.float32)
        out_ref[s] = outv + outb_ref[...]


def kernel(features, in_w, in_b, out_w, out_b):
    bs, P, D = features.shape
    O = out_w.shape[0]
    seeds = jax.random.uniform(jax.random.key(42), (bs, P), dtype=jnp.float32)
    log_seeds = jnp.log(jnp.clip(seeds, _EPS, None)).reshape(bs, 1, P)
    in_b2 = in_b.reshape(1, D)
    out_b2 = out_b.reshape(1, O)
    return pl.pallas_call(
        _sb_kernel,
        grid=(bs // _S,),
        in_specs=[
            pl.BlockSpec((_S, P, D), lambda b: (b, 0, 0)),
            pl.BlockSpec((D, D), lambda b: (0, 0)),
            pl.BlockSpec((1, D), lambda b: (0, 0)),
            pl.BlockSpec((O, D), lambda b: (0, 0)),
            pl.BlockSpec((1, O), lambda b: (0, 0)),
            pl.BlockSpec((_S, 1, P), lambda b: (b, 0, 0)),
        ],
        out_specs=pl.BlockSpec((_S, _N_SLOTS, O), lambda b: (b, 0, 0)),
        out_shape=jax.ShapeDtypeStruct((bs, _N_SLOTS, O), jnp.float32),
        scratch_shapes=[
            pltpu.VMEM((_S, P, P), jnp.float32),
            pltpu.VMEM((_S, _N_SLOTS, P), jnp.float32),
        ],
        compiler_params=pltpu.CompilerParams(
            dimension_semantics=("parallel",)),
    )(features, in_w, in_b2, out_w, out_b2, log_seeds)


# speedup vs baseline: 1.8683x; 1.2907x over previous
import numpy as np
import jax
import jax.numpy as jnp
from jax.experimental import pallas as pl
from jax.experimental.pallas import tpu as pltpu

_EPS = 1e-08
_LOG_EPS = float(np.log(1e-08))
_N_SLOTS = 16
_S = 4


def _sb_kernel(feat_ref, inw_ref, inb_ref, outw_ref, outb_ref, lseed_ref,
               out_ref, *scratch):
    g_refs = scratch[:_S]
    mask_refs = scratch[_S:]
    P = feat_ref.shape[1]
    D = feat_ref.shape[2]

    f_all = feat_ref[...].reshape(_S * P, D)
    raw = jax.lax.dot_general(f_all.astype(jnp.bfloat16),
                              inw_ref[...].astype(jnp.bfloat16),
                              (((1,), (1,)), ((), ())),
                              preferred_element_type=jnp.float32)
    raw = raw + inb_ref[...]
    norm = jnp.sqrt(jnp.sum(raw * raw, axis=-1, keepdims=True))
    proj = raw / jnp.maximum(norm, 1e-12)

    hi = proj.astype(jnp.bfloat16)
    lo = (proj - hi.astype(jnp.float32)).astype(jnp.bfloat16)
    u = jnp.concatenate([hi, lo, hi, lo], axis=1)
    w = jnp.concatenate([hi, lo, lo, hi], axis=1)
    for s in range(_S):
        g_refs[s][...] = jax.lax.dot_general(
            u[s * P:(s + 1) * P], w[s * P:(s + 1) * P],
            (((1,), (1,)), ((), ())), preferred_element_type=jnp.float32)

    lseeds = lseed_ref[...].reshape(_S, P)
    ids = jax.lax.broadcasted_iota(jnp.int32, (_S, P), 1)
    scope = jnp.zeros((_S, P), jnp.float32)

    for k in range(_N_SLOTS):
        v = scope + lseeds
        mx = jnp.max(v, axis=1, keepdims=True)
        idxs = jnp.min(jnp.where(v == mx, ids, P), axis=1)
        rows = [g_refs[s][pl.ds(idxs[s], 1), :] for s in range(_S)]
        grow = jnp.concatenate(rows, axis=0)
        dists = 2.0 - 2.0 * grow
        log_alpha = jnp.maximum(-dists, _LOG_EPS)
        mask = jnp.exp(scope + log_alpha)
        for s in range(_S):
            mask_refs[s][k:k + 1, :] = mask[s:s + 1, :]
        scope = scope + jnp.log(jnp.maximum(1.0 - jnp.exp(log_alpha), _EPS))

    outw_bf = outw_ref[...].astype(jnp.bfloat16)
    f_bf = f_all.astype(jnp.bfloat16)
    for s in range(_S):
        masks = mask_refs[s][...]
        slots = jax.lax.dot_general(masks.astype(jnp.bfloat16),
                                    f_bf[s * P:(s + 1) * P],
                                    (((1,), (0,)), ((), ())),
                                    preferred_element_type=jnp.float32)
        msum = jnp.sum(masks, axis=1, keepdims=True)
        slots = slots / jnp.maximum(msum, _EPS)
        outv = jax.lax.dot_general(slots.astype(jnp.bfloat16), outw_bf,
                                   (((1,), (1,)), ((), ())),
                                   preferred_element_type=jnp.float32)
        out_ref[s] = outv + outb_ref[...]


def kernel(features, in_w, in_b, out_w, out_b):
    bs, P, D = features.shape
    O = out_w.shape[0]
    seeds = jax.random.uniform(jax.random.key(42), (bs, P), dtype=jnp.float32)
    log_seeds = jnp.log(jnp.clip(seeds, _EPS, None)).reshape(bs, 1, P)
    in_b2 = in_b.reshape(1, D)
    out_b2 = out_b.reshape(1, O)
    return pl.pallas_call(
        _sb_kernel,
        grid=(bs // _S,),
        in_specs=[
            pl.BlockSpec((_S, P, D), lambda b: (b, 0, 0)),
            pl.BlockSpec((D, D), lambda b: (0, 0)),
            pl.BlockSpec((1, D), lambda b: (0, 0)),
            pl.BlockSpec((O, D), lambda b: (0, 0)),
            pl.BlockSpec((1, O), lambda b: (0, 0)),
            pl.BlockSpec((_S, 1, P), lambda b: (b, 0, 0)),
        ],
        out_specs=pl.BlockSpec((_S, _N_SLOTS, O), lambda b: (b, 0, 0)),
        out_shape=jax.ShapeDtypeStruct((bs, _N_SLOTS, O), jnp.float32),
        scratch_shapes=(
            [pltpu.VMEM((P, P), jnp.float32) for _ in range(_S)]
            + [pltpu.VMEM((_N_SLOTS, P), jnp.float32) for _ in range(_S)]),
        compiler_params=pltpu.CompilerParams(
            dimension_semantics=("parallel",)),
    )(features, in_w, in_b2, out_w, out_b2, log_seeds)


# speedup vs baseline: 1.9086x; 1.0216x over previous
import numpy as np
import jax
import jax.numpy as jnp
from jax.experimental import pallas as pl
from jax.experimental.pallas import tpu as pltpu

_EPS = 1e-08
_LOG_EPS = float(np.log(1e-08))
_N_SLOTS = 16
_S = 2
_G = 2


def _sb_kernel(feat_ref, inw_ref, inb_ref, outw_ref, outb_ref, lseed_ref,
               out_ref, *scratch):
    g_refs = scratch[:_G * _S]
    mask_refs = scratch[_G * _S:]
    P = feat_ref.shape[1]
    D = feat_ref.shape[2]
    inw_bf = inw_ref[...].astype(jnp.bfloat16)

    f_bfs = []

    def phase_a(g):
        base = g * _S
        f_all = feat_ref[...][base:base + _S].reshape(_S * P, D)
        f_bf = f_all.astype(jnp.bfloat16)
        f_bfs.append(f_bf)
        raw = jax.lax.dot_general(f_bf, inw_bf, (((1,), (1,)), ((), ())),
                                  preferred_element_type=jnp.float32)
        raw = raw + inb_ref[...]
        norm = jnp.sqrt(jnp.sum(raw * raw, axis=-1, keepdims=True))
        proj = raw / jnp.maximum(norm, 1e-12)
        hi = proj.astype(jnp.bfloat16)
        lo = (proj - hi.astype(jnp.float32)).astype(jnp.bfloat16)
        u = jnp.concatenate([hi, lo, hi, lo], axis=1)
        w = jnp.concatenate([hi, lo, lo, hi], axis=1)
        for s in range(_S):
            g_refs[base + s][...] = jax.lax.dot_general(
                u[s * P:(s + 1) * P], w[s * P:(s + 1) * P],
                (((1,), (1,)), ((), ())), preferred_element_type=jnp.float32)

    def phase_b(g):
        base = g * _S
        lseeds = lseed_ref[...][base:base + _S].reshape(_S, P)
        ids = jax.lax.broadcasted_iota(jnp.int32, (_S, P), 1)
        scope = jnp.zeros((_S, P), jnp.float32)
        for k in range(_N_SLOTS):
            v = scope + lseeds
            mx = jnp.max(v, axis=1, keepdims=True)
            idxs = jnp.min(jnp.where(v == mx, ids, P), axis=1)
            rows = [g_refs[base + s][pl.ds(idxs[s], 1), :] for s in range(_S)]
            grow = jnp.concatenate(rows, axis=0)
            dists = 2.0 - 2.0 * grow
            log_alpha = jnp.maximum(-dists, _LOG_EPS)
            mask = jnp.exp(scope + log_alpha)
            for s in range(_S):
                mask_refs[g][s, k:k + 1, :] = mask[s:s + 1, :]
            scope = scope + jnp.log(
                jnp.maximum(1.0 - jnp.exp(log_alpha), _EPS))

        outw_bf = outw_ref[...].astype(jnp.bfloat16)
        f_bf = f_bfs[g]
        for s in range(_S):
            masks = mask_refs[g][s]
            slots = jax.lax.dot_general(masks.astype(jnp.bfloat16),
                                        f_bf[s * P:(s + 1) * P],
                                        (((1,), (0,)), ((), ())),
                                        preferred_element_type=jnp.float32)
            msum = jnp.sum(masks, axis=1, keepdims=True)
            slots = slots / jnp.maximum(msum, _EPS)
            outv = jax.lax.dot_general(slots.astype(jnp.bfloat16), outw_bf,
                                       (((1,), (1,)), ((), ())),
                                       preferred_element_type=jnp.float32)
            out_ref[base + s] = outv + outb_ref[...]

    phase_a(0)
    phase_a(1)
    phase_b(0)
    phase_b(1)


def kernel(features, in_w, in_b, out_w, out_b):
    bs, P, D = features.shape
    O = out_w.shape[0]
    step = _G * _S
    seeds = jax.random.uniform(jax.random.key(42), (bs, P), dtype=jnp.float32)
    log_seeds = jnp.log(jnp.clip(seeds, _EPS, None)).reshape(bs, 1, P)
    in_b2 = in_b.reshape(1, D)
    out_b2 = out_b.reshape(1, O)
    return pl.pallas_call(
        _sb_kernel,
        grid=(bs // step,),
        in_specs=[
            pl.BlockSpec((step, P, D), lambda b: (b, 0, 0)),
            pl.BlockSpec((D, D), lambda b: (0, 0)),
            pl.BlockSpec((1, D), lambda b: (0, 0)),
            pl.BlockSpec((O, D), lambda b: (0, 0)),
            pl.BlockSpec((1, O), lambda b: (0, 0)),
            pl.BlockSpec((step, 1, P), lambda b: (b, 0, 0)),
        ],
        out_specs=pl.BlockSpec((step, _N_SLOTS, O), lambda b: (b, 0, 0)),
        out_shape=jax.ShapeDtypeStruct((bs, _N_SLOTS, O), jnp.float32),
        scratch_shapes=(
            [pltpu.VMEM((P, P), jnp.float32) for _ in range(_G * _S)]
            + [pltpu.VMEM((_S, _N_SLOTS, P), jnp.float32)
               for _ in range(_G)]),
        compiler_params=pltpu.CompilerParams(
            dimension_semantics=("parallel",)),
    )(features, in_w, in_b2, out_w, out_b2, log_seeds)


# speedup vs baseline: 1.9544x; 1.0240x over previous
import numpy as np
import jax
import jax.numpy as jnp
from jax.experimental import pallas as pl
from jax.experimental.pallas import tpu as pltpu

_EPS = 1e-08
_LOG_EPS = float(np.log(1e-08))
_N_SLOTS = 16
_S = 2
_G = 2


def _sb_kernel(feat_ref, inw_ref, inb_ref, outw_ref, outb_ref, lseed_ref,
               out_ref, *scratch):
    g_refs = scratch[:_G * _S]
    mask_refs = scratch[_G * _S:]
    P = feat_ref.shape[1]
    D = feat_ref.shape[2]
    inw_bf = inw_ref[...].astype(jnp.bfloat16)

    f_bfs = []

    def phase_a(g):
        base = g * _S
        f_all = feat_ref[...][base:base + _S].reshape(_S * P, D)
        f_bf = f_all.astype(jnp.bfloat16)
        f_bfs.append(f_bf)
        raw = jax.lax.dot_general(f_bf, inw_bf, (((1,), (1,)), ((), ())),
                                  preferred_element_type=jnp.float32)
        raw = raw + inb_ref[...]
        norm = jnp.sqrt(jnp.sum(raw * raw, axis=-1, keepdims=True))
        proj = raw / jnp.maximum(norm, 1e-12)
        hi = proj.astype(jnp.bfloat16)
        lo = (proj - hi.astype(jnp.float32)).astype(jnp.bfloat16)
        u = jnp.concatenate([hi, lo, hi, lo], axis=1)
        w = jnp.concatenate([hi, lo, lo, hi], axis=1)
        for s in range(_S):
            g_refs[base + s][...] = jax.lax.dot_general(
                u[s * P:(s + 1) * P], w[s * P:(s + 1) * P],
                (((1,), (1,)), ((), ())), preferred_element_type=jnp.float32)

    def loop_all():
        n = _G * _S
        seeds = lseed_ref[...].reshape(n, P)
        ids = jax.lax.broadcasted_iota(jnp.int32, (n, P), 1)
        escope = jnp.ones((n, P), jnp.float32)
        for k in range(_N_SLOTS):
            v = escope * seeds
            mx = jnp.max(v, axis=1, keepdims=True)
            idxs = jnp.min(jnp.where(v == mx, ids, P), axis=1)
            rows = [g_refs[s][pl.ds(idxs[s], 1), :] for s in range(n)]
            grow = jnp.concatenate(rows, axis=0)
            dists = 2.0 - 2.0 * grow
            log_alpha = jnp.maximum(-dists, _LOG_EPS)
            alpha = jnp.exp(log_alpha)
            mask = escope * alpha
            for s in range(n):
                mask_refs[s // _S][s % _S, k:k + 1, :] = mask[s:s + 1, :]
            escope = escope * jnp.maximum(1.0 - alpha, _EPS)

    def pool(g):
        base = g * _S
        outw_bf = outw_ref[...].astype(jnp.bfloat16)
        f_bf = f_bfs[g]
        for s in range(_S):
            masks = mask_refs[g][s]
            slots = jax.lax.dot_general(masks.astype(jnp.bfloat16),
                                        f_bf[s * P:(s + 1) * P],
                                        (((1,), (0,)), ((), ())),
                                        preferred_element_type=jnp.float32)
            msum = jnp.sum(masks, axis=1, keepdims=True)
            slots = slots / jnp.maximum(msum, _EPS)
            outv = jax.lax.dot_general(slots.astype(jnp.bfloat16), outw_bf,
                                       (((1,), (1,)), ((), ())),
                                       preferred_element_type=jnp.float32)
            out_ref[base + s] = outv + outb_ref[...]

    phase_a(0)
    phase_a(1)
    loop_all()
    pool(0)
    pool(1)


def kernel(features, in_w, in_b, out_w, out_b):
    bs, P, D = features.shape
    O = out_w.shape[0]
    step = _G * _S
    seeds = jax.random.uniform(jax.random.key(42), (bs, P), dtype=jnp.float32)
    seeds_c = jnp.clip(seeds, _EPS, None).reshape(bs, 1, P)
    in_b2 = in_b.reshape(1, D)
    out_b2 = out_b.reshape(1, O)
    return pl.pallas_call(
        _sb_kernel,
        grid=(bs // step,),
        in_specs=[
            pl.BlockSpec((step, P, D), lambda b: (b, 0, 0)),
            pl.BlockSpec((D, D), lambda b: (0, 0)),
            pl.BlockSpec((1, D), lambda b: (0, 0)),
            pl.BlockSpec((O, D), lambda b: (0, 0)),
            pl.BlockSpec((1, O), lambda b: (0, 0)),
            pl.BlockSpec((step, 1, P), lambda b: (b, 0, 0)),
        ],
        out_specs=pl.BlockSpec((step, _N_SLOTS, O), lambda b: (b, 0, 0)),
        out_shape=jax.ShapeDtypeStruct((bs, _N_SLOTS, O), jnp.float32),
        scratch_shapes=(
            [pltpu.VMEM((P, P), jnp.float32) for _ in range(_G * _S)]
            + [pltpu.VMEM((_S, _N_SLOTS, P), jnp.float32)
               for _ in range(_G)]),
        compiler_params=pltpu.CompilerParams(
            dimension_semantics=("parallel",)),
    )(features, in_w, in_b2, out_w, out_b2, seeds_c)
